# baseline (speedup 1.0000x reference)
import jax
import jax.numpy as jnp
from jax import lax
from jax.experimental import pallas as pl
from jax.experimental.pallas import tpu as pltpu
from jax.experimental.pallas import tpu_sc as plsc

N = 10000
E = 320000
DF = 128
DH = 16
DO = 2
DP = 8

NC = 2
NS = 16
CH = 128
NCH = -(-E // (NC * NS * CH))
NCH = -(-NCH // 3) * 3
EP = NC * NS * NCH * CH
NROWS = 10240
RPT = NROWS // NS

_sc_mesh = plsc.VectorSubcoreMesh(core_axis_name="c", subcore_axis_name="s")
_sc_params = pltpu.CompilerParams(use_tc_tiling_on_sc=False)


def _deg_body(idx_hbm, zeros_hbm, out_hbm, acc, idxb, ones_v, sem):
    c = lax.axis_index("c")
    s = lax.axis_index("s")
    for i in range(CH // 16):
        ones_v[pl.ds(i * 16, 16)] = jnp.full((16,), 1.0, jnp.float32)
    pltpu.async_copy(idx_hbm.at[c, s], idxb, sem)
    pltpu.sync_copy(zeros_hbm.at[pl.ds(s * RPT, RPT)],
                    acc.at[pl.ds(s * RPT, RPT)])
    pltpu.make_async_copy(idx_hbm.at[c, s], idxb, sem).wait()
    plsc.subcore_barrier()

    def chunk(j, carry):
        pltpu.sync_copy(ones_v, acc.at[idxb.at[2 * j + 1]], add=True)
        return carry

    lax.fori_loop(0, NCH, chunk, 0)
    plsc.subcore_barrier()
    pltpu.sync_copy(acc.at[pl.ds(s * RPT, RPT)],
                    out_hbm.at[c].at[pl.ds(s * RPT, RPT)])


_deg_call = pl.kernel(
    _deg_body,
    out_type=jax.ShapeDtypeStruct((NC, NROWS), jnp.float32),
    mesh=_sc_mesh,
    scratch_types=[
        pltpu.VMEM_SHARED((NROWS,), jnp.float32),
        pltpu.VMEM((2 * NCH, CH), jnp.int32),
        pltpu.VMEM((CH,), jnp.float32),
        pltpu.SemaphoreType.DMA,
    ],
    compiler_params=_sc_params,
)


def _make_agg_call(width):
    def body(idx_hbm, table_hbm, zeros_hbm, out_hbm,
             acc, idxb, rows0, rows1, rows2, sem_g0, sem_g1, sem_g2):
        c = lax.axis_index("c")
        s = lax.axis_index("s")
        pltpu.async_copy(idx_hbm.at[c, s], idxb, sem_g0)
        pltpu.sync_copy(zeros_hbm.at[pl.ds(s * RPT, RPT)],
                        acc.at[pl.ds(s * RPT, RPT)])
        pltpu.make_async_copy(idx_hbm.at[c, s], idxb, sem_g0).wait()
        plsc.subcore_barrier()

        pltpu.async_copy(table_hbm.at[idxb.at[0]], rows0, sem_g0)
        pltpu.async_copy(table_hbm.at[idxb.at[2]], rows1, sem_g1)

        def triple(jj, carry):
            j2 = 6 * jj
            pltpu.async_copy(table_hbm.at[idxb.at[j2 + 4]], rows2, sem_g2)
            pltpu.make_async_copy(table_hbm.at[idxb.at[0]], rows0,
                                  sem_g0).wait()
            pltpu.sync_copy(rows0, acc.at[idxb.at[j2 + 1]], add=True)
            ja = jnp.minimum(j2 + 6, 2 * NCH - 2)
            pltpu.async_copy(table_hbm.at[idxb.at[ja]], rows0, sem_g0)
            pltpu.make_async_copy(table_hbm.at[idxb.at[0]], rows1,
                                  sem_g1).wait()
            pltpu.sync_copy(rows1, acc.at[idxb.at[j2 + 3]], add=True)
            jb = jnp.minimum(j2 + 8, 2 * NCH - 2)
            pltpu.async_copy(table_hbm.at[idxb.at[jb]], rows1, sem_g1)
            pltpu.make_async_copy(table_hbm.at[idxb.at[0]], rows2,
                                  sem_g2).wait()
            pltpu.sync_copy(rows2, acc.at[idxb.at[j2 + 5]], add=True)
            return carry

        lax.fori_loop(0, NCH // 3, triple, 0)
        pltpu.make_async_copy(table_hbm.at[idxb.at[0]], rows0, sem_g0).wait()
        pltpu.make_async_copy(table_hbm.at[idxb.at[0]], rows1, sem_g1).wait()
        plsc.subcore_barrier()
        pltpu.sync_copy(acc.at[pl.ds(s * RPT, RPT)],
                        out_hbm.at[c].at[pl.ds(s * RPT, RPT)])

    return pl.kernel(
        body,
        out_type=jax.ShapeDtypeStruct((NC, NROWS, width), jnp.float32),
        mesh=_sc_mesh,
        scratch_types=[
            pltpu.VMEM_SHARED((NROWS, width), jnp.float32),
            pltpu.VMEM((2 * NCH, CH), jnp.int32),
            pltpu.VMEM((CH, width), jnp.float32),
            pltpu.VMEM((CH, width), jnp.float32),
            pltpu.VMEM((CH, width), jnp.float32),
            pltpu.SemaphoreType.DMA,
            pltpu.SemaphoreType.DMA,
            pltpu.SemaphoreType.DMA,
        ],
        compiler_params=_sc_params,
    )


_agg_call = _make_agg_call(DH)
_agg2_call = _make_agg_call(DP)


def _tc_b_body(x_ref, w1_ref, degp_ref, hs1_ref, dis_ref):
    h = jnp.dot(x_ref[...], w1_ref[...], preferred_element_type=jnp.float32)
    deg = degp_ref[:, 0:1] + degp_ref[:, 1:2] + 1.0
    dis = lax.rsqrt(deg)
    dis_ref[...] = dis
    hs1_ref[...] = h * dis[0:N, :]


_tc_b = pl.pallas_call(
    _tc_b_body,
    out_shape=(
        jax.ShapeDtypeStruct((N, DH), jnp.float32),
        jax.ShapeDtypeStruct((NROWS, 1), jnp.float32),
    ),
)


def _tc_d_body(aggp_ref, hs1_ref, dis_ref, b1_ref, w2_ref, h1s2_ref):
    a = aggp_ref[0, 0:N, :] + aggp_ref[1, 0:N, :]
    dis = dis_ref[0:N, :]
    pre = (a + hs1_ref[...]) * dis + b1_ref[...]
    h1 = jnp.maximum(pre, 0.0)
    h1s2_ref[...] = jnp.dot(h1 * dis, w2_ref[...],
                            preferred_element_type=jnp.float32)


_tc_d = pl.pallas_call(
    _tc_d_body,
    out_shape=jax.ShapeDtypeStruct((N, DP), jnp.float32),
)


def _tc_f_body(aggp_ref, h1s2_ref, dis_ref, b2_ref, out_ref):
    v = (aggp_ref[0, 0:N, :] + aggp_ref[1, 0:N, :] + h1s2_ref[...]) \
        * dis_ref[0:N, :]
    cl = v[:, 0:DO] + b2_ref[...]
    m = jnp.max(cl, axis=1, keepdims=True)
    e = jnp.exp(cl - m)
    lse = m + jnp.log(jnp.sum(e, axis=1, keepdims=True))
    out_ref[...] = cl - lse


_tc_f = pl.pallas_call(
    _tc_f_body,
    out_shape=jax.ShapeDtypeStruct((N, DO), jnp.float32),
)


def kernel(x, edge_index, W1, b1, W2, b2):
    ei = edge_index.astype(jnp.int32)
    pad = EP - E
    srcp = jnp.concatenate([ei[0], jnp.zeros((pad,), jnp.int32)])
    srcp = srcp.reshape(NC, NS, NCH, CH)
    dstp = jnp.concatenate([ei[1], jnp.full((pad,), N, jnp.int32)])
    dstp = dstp.reshape(NC, NS, NCH, CH)
    pidx = jnp.stack([srcp, dstp], axis=3).reshape(NC, NS, 2 * NCH, CH)
    zeros1 = jnp.zeros((NROWS,), jnp.float32)
    zeros2 = jnp.zeros((NROWS, DH), jnp.float32)
    zeros3 = jnp.zeros((NROWS, DP), jnp.float32)
    W2p = jnp.zeros((DH, DP), jnp.float32).at[:, 0:DO].set(W2)

    degp = _deg_call(pidx, zeros1)
    hs1, dis = _tc_b(x, W1, degp.T)
    agg1 = _agg_call(pidx, hs1, zeros2)
    h1s2 = _tc_d(agg1, hs1, dis, b1.reshape(1, DH), W2p)
    agg2 = _agg2_call(pidx, h1s2, zeros3)
    return _tc_f(agg2, h1s2, dis, b2.reshape(1, DO))


# speedup vs baseline: 1.1776x; 1.1776x over previous
import jax
import jax.numpy as jnp
from jax import lax
from jax.experimental import pallas as pl
from jax.experimental.pallas import tpu as pltpu
from jax.experimental.pallas import tpu_sc as plsc

N = 10000
E = 320000
DF = 128
DH = 16
DO = 2
DP = 8

NC = 2
NS = 16
CH = 128
NCH = -(-E // (NC * NS * CH))
NCH += NCH % 2
EP = NC * NS * NCH * CH
NROWS = 10240
RPT = NROWS // NS

_sc_mesh = plsc.VectorSubcoreMesh(core_axis_name="c", subcore_axis_name="s")
_sc_params = pltpu.CompilerParams(use_tc_tiling_on_sc=False)


def _deg_body(idx_hbm, zeros_hbm, out_hbm, acc, idxb, ones_v, sem):
    c = lax.axis_index("c")
    s = lax.axis_index("s")
    for i in range(CH // 16):
        ones_v[pl.ds(i * 16, 16)] = jnp.full((16,), 1.0, jnp.float32)
    pltpu.async_copy(idx_hbm.at[c, s], idxb, sem)
    pltpu.sync_copy(zeros_hbm.at[pl.ds(s * RPT, RPT)],
                    acc.at[pl.ds(s * RPT, RPT)])
    pltpu.make_async_copy(idx_hbm.at[c, s], idxb, sem).wait()
    plsc.subcore_barrier()

    def chunk(j, carry):
        pltpu.sync_copy(ones_v, acc.at[idxb.at[2 * j + 1]], add=True)
        return carry

    lax.fori_loop(0, NCH, chunk, 0)
    plsc.subcore_barrier()
    pltpu.sync_copy(acc.at[pl.ds(s * RPT, RPT)],
                    out_hbm.at[c].at[pl.ds(s * RPT, RPT)])


_deg_call = pl.kernel(
    _deg_body,
    out_type=jax.ShapeDtypeStruct((NC, NROWS), jnp.float32),
    mesh=_sc_mesh,
    scratch_types=[
        pltpu.VMEM_SHARED((NROWS,), jnp.float32),
        pltpu.VMEM((2 * NCH, CH), jnp.int32),
        pltpu.VMEM((CH,), jnp.float32),
        pltpu.SemaphoreType.DMA,
    ],
    compiler_params=_sc_params,
)


def _make_agg_call(width):
    def body(idx_hbm, table_hbm, zeros_hbm, out_hbm,
             acc, idxb, rows0, rows1, sem_g0, sem_g1):
        c = lax.axis_index("c")
        s = lax.axis_index("s")
        pltpu.async_copy(idx_hbm.at[c, s], idxb, sem_g0)
        pltpu.sync_copy(zeros_hbm.at[pl.ds(s * RPT, RPT)],
                        acc.at[pl.ds(s * RPT, RPT)])
        pltpu.make_async_copy(idx_hbm.at[c, s], idxb, sem_g0).wait()
        plsc.subcore_barrier()

        pltpu.async_copy(table_hbm.at[idxb.at[0]], rows0, sem_g0)

        def pair(jj, carry):
            j2 = 4 * jj
            pltpu.async_copy(table_hbm.at[idxb.at[j2 + 2]], rows1, sem_g1)
            pltpu.make_async_copy(table_hbm.at[idxb.at[0]], rows0,
                                  sem_g0).wait()
            pltpu.sync_copy(rows0, acc.at[idxb.at[j2 + 1]], add=True)
            jn = jnp.minimum(j2 + 4, 2 * NCH - 2)
            pltpu.async_copy(table_hbm.at[idxb.at[jn]], rows0, sem_g0)
            pltpu.make_async_copy(table_hbm.at[idxb.at[0]], rows1,
                                  sem_g1).wait()
            pltpu.sync_copy(rows1, acc.at[idxb.at[j2 + 3]], add=True)
            return carry

        lax.fori_loop(0, NCH // 2, pair, 0)
        pltpu.make_async_copy(table_hbm.at[idxb.at[0]], rows0, sem_g0).wait()
        plsc.subcore_barrier()
        pltpu.sync_copy(acc.at[pl.ds(s * RPT, RPT)],
                        out_hbm.at[c].at[pl.ds(s * RPT, RPT)])

    return pl.kernel(
        body,
        out_type=jax.ShapeDtypeStruct((NC, NROWS, width), jnp.float32),
        mesh=_sc_mesh,
        scratch_types=[
            pltpu.VMEM_SHARED((NROWS, width), jnp.float32),
            pltpu.VMEM((2 * NCH, CH), jnp.int32),
            pltpu.VMEM((CH, width), jnp.float32),
            pltpu.VMEM((CH, width), jnp.float32),
            pltpu.SemaphoreType.DMA,
            pltpu.SemaphoreType.DMA,
        ],
        compiler_params=_sc_params,
    )


_agg_call = _make_agg_call(DH)
_agg2_call = _make_agg_call(DP)


def _tc_b_body(x_ref, w1_ref, degp_ref, hs1_ref, dis_ref):
    h = jnp.dot(x_ref[...], w1_ref[...], preferred_element_type=jnp.float32)
    deg = degp_ref[:, 0:1] + degp_ref[:, 1:2] + 1.0
    dis = lax.rsqrt(deg)
    dis_ref[...] = dis
    hs1_ref[...] = h * dis[0:N, :]


_tc_b = pl.pallas_call(
    _tc_b_body,
    out_shape=(
        jax.ShapeDtypeStruct((N, DH), jnp.float32),
        jax.ShapeDtypeStruct((NROWS, 1), jnp.float32),
    ),
)


def _tc_d_body(aggp_ref, hs1_ref, dis_ref, b1_ref, w2_ref, h1s2_ref):
    a = aggp_ref[0, 0:N, :] + aggp_ref[1, 0:N, :]
    dis = dis_ref[0:N, :]
    pre = (a + hs1_ref[...]) * dis + b1_ref[...]
    h1 = jnp.maximum(pre, 0.0)
    h1s2_ref[...] = jnp.dot(h1 * dis, w2_ref[...],
                            preferred_element_type=jnp.float32)


_tc_d = pl.pallas_call(
    _tc_d_body,
    out_shape=jax.ShapeDtypeStruct((N, DP), jnp.float32),
)


def _tc_f_body(aggp_ref, h1s2_ref, dis_ref, b2_ref, out_ref):
    v = (aggp_ref[0, 0:N, :] + aggp_ref[1, 0:N, :] + h1s2_ref[...]) \
        * dis_ref[0:N, :]
    cl = v[:, 0:DO] + b2_ref[...]
    m = jnp.max(cl, axis=1, keepdims=True)
    e = jnp.exp(cl - m)
    lse = m + jnp.log(jnp.sum(e, axis=1, keepdims=True))
    out_ref[...] = cl - lse


_tc_f = pl.pallas_call(
    _tc_f_body,
    out_shape=jax.ShapeDtypeStruct((N, DO), jnp.float32),
)


def kernel(x, edge_index, W1, b1, W2, b2):
    ei = edge_index.astype(jnp.int32)
    pad = EP - E
    srcp = jnp.concatenate([ei[0], jnp.zeros((pad,), jnp.int32)])
    srcp = srcp.reshape(NC, NS, NCH, CH)
    dstp = jnp.concatenate([ei[1], jnp.full((pad,), N, jnp.int32)])
    dstp = dstp.reshape(NC, NS, NCH, CH)
    pidx = jnp.stack([srcp, dstp], axis=3).reshape(NC, NS, 2 * NCH, CH)
    zeros1 = jnp.zeros((NROWS,), jnp.float32)
    zeros2 = jnp.zeros((NROWS, DH), jnp.float32)
    zeros3 = jnp.zeros((NROWS, DP), jnp.float32)
    W2p = jnp.zeros((DH, DP), jnp.float32).at[:, 0:DO].set(W2)

    degp = _deg_call(pidx, zeros1)
    hs1, dis = _tc_b(x, W1, degp.T)
    agg1 = _agg_call(pidx, hs1, zeros2)
    h1s2 = _tc_d(agg1, hs1, dis, b1.reshape(1, DH), W2p)
    agg2 = _agg2_call(pidx, h1s2, zeros3)
    return _tc_f(agg2, h1s2, dis, b2.reshape(1, DO))
